# Initial kernel scaffold; baseline (speedup 1.0000x reference)
import jax
import jax.numpy as jnp
from jax.experimental import pallas as pl


def kernel(x, wts):
    raise NotImplementedError("write your pallas kernel here")


# speedup vs baseline: 1.0228x; 1.0228x over previous
import functools

import jax
import jax.numpy as jnp
from jax import lax
from jax.experimental import pallas as pl
from jax.experimental.pallas import tpu as pltpu
from jax.experimental.pallas import tpu_sc as plsc

INPUT_DIM = 1000000
EMBED_DIM = 32
B = 16384
T = 50

N = B * T
NC, NS = 2, 16
NW = NC * NS
CHUNK = 128
N_PER_W = N // NW
N_CHUNKS = N_PER_W // CHUNK


def _make_kernel():
    mesh = plsc.VectorSubcoreMesh(core_axis_name="c", subcore_axis_name="s")

    @functools.partial(
        pl.kernel,
        out_type=jax.ShapeDtypeStruct((N, EMBED_DIM), jnp.float32),
        mesh=mesh,
        scratch_types=[
            pltpu.VMEM((N_CHUNKS, CHUNK), jnp.int32),
            pltpu.VMEM((CHUNK, EMBED_DIM), jnp.float32),
            pltpu.SemaphoreType.DMA,
        ],
        compiler_params=pltpu.CompilerParams(use_tc_tiling_on_sc=False),
    )
    def emb(idx_hbm, table_hbm, out_hbm, idx_v, rows_v, sem):
        wid = lax.axis_index("s") * NC + lax.axis_index("c")
        base = wid * N_PER_W
        pltpu.sync_copy(idx_hbm.at[pl.ds(wid * N_CHUNKS, N_CHUNKS)], idx_v)

        @pl.loop(0, N_CHUNKS)
        def _chunk(j):
            pltpu.async_copy(table_hbm.at[idx_v.at[j]], rows_v, sem).wait()
            pltpu.sync_copy(rows_v, out_hbm.at[pl.ds(base + j * CHUNK, CHUNK)])

    return emb


_emb_kernel = _make_kernel()


def kernel(x, wts):
    idx = x.reshape(N // CHUNK, CHUNK)
    out = _emb_kernel(idx, wts)
    return out.reshape(B, T, EMBED_DIM)


# speedup vs baseline: 1.1132x; 1.0884x over previous
import functools

import jax
import jax.numpy as jnp
from jax import lax
from jax.experimental import pallas as pl
from jax.experimental.pallas import tpu as pltpu
from jax.experimental.pallas import tpu_sc as plsc

INPUT_DIM = 1000000
EMBED_DIM = 32
B = 16384
T = 50

N = B * T
NC, NS = 2, 16
NW = NC * NS
CHUNK = 128
N_PER_W = N // NW
N_CHUNKS = N_PER_W // CHUNK
G = 10
GROUP = G * CHUNK
NG = N_CHUNKS // G
NPAIR = NG // 2


def _make_kernel():
    mesh = plsc.VectorSubcoreMesh(core_axis_name="c", subcore_axis_name="s")

    @functools.partial(
        pl.kernel,
        out_type=jax.ShapeDtypeStruct((N, EMBED_DIM), jnp.float32),
        mesh=mesh,
        scratch_types=[
            pltpu.VMEM((N_CHUNKS, CHUNK), jnp.int32),
            pltpu.VMEM((GROUP, EMBED_DIM), jnp.float32),
            pltpu.VMEM((GROUP, EMBED_DIM), jnp.float32),
            pltpu.SemaphoreType.DMA,
            pltpu.SemaphoreType.DMA,
        ],
        compiler_params=pltpu.CompilerParams(use_tc_tiling_on_sc=False),
    )
    def emb(idx_hbm, table_hbm, out_hbm, idx_v, buf0, buf1, gsem, ssem):
        wid = lax.axis_index("s") * NC + lax.axis_index("c")
        base = wid * N_PER_W
        pltpu.sync_copy(idx_hbm.at[pl.ds(wid * N_CHUNKS, N_CHUNKS)], idx_v)

        def fire_gathers(g, buf):
            @pl.loop(0, G)
            def _(b):
                pltpu.make_async_copy(
                    table_hbm.at[idx_v.at[g * G + b]],
                    buf.at[pl.ds(b * CHUNK, CHUNK)],
                    gsem,
                ).start()

        def wait_gathers(buf):
            pltpu.make_async_copy(table_hbm.at[pl.ds(0, GROUP)], buf, gsem).wait()

        def out_slice(g):
            return out_hbm.at[pl.ds(base + g * GROUP, GROUP)]

        def fire_scatter(g, buf):
            pltpu.make_async_copy(buf, out_slice(g), ssem).start()

        def wait_scatter(g, buf):
            pltpu.make_async_copy(buf, out_slice(g), ssem).wait()

        fire_gathers(0, buf0)

        @pl.loop(0, NPAIR)
        def _(p):
            g0 = 2 * p

            @pl.when(p > 0)
            def _():
                wait_scatter(g0 - 1, buf1)

            fire_gathers(g0 + 1, buf1)
            wait_gathers(buf0)
            fire_scatter(g0, buf0)
            wait_scatter(g0, buf0)

            @pl.when(p < NPAIR - 1)
            def _():
                fire_gathers(g0 + 2, buf0)

            wait_gathers(buf1)
            fire_scatter(g0 + 1, buf1)

        wait_scatter(NG - 1, buf1)

    return emb


_emb_kernel = _make_kernel()


def kernel(x, wts):
    idx = x.reshape(N // CHUNK, CHUNK)
    out = _emb_kernel(idx, wts)
    return out.reshape(B, T, EMBED_DIM)


# speedup vs baseline: 1.3658x; 1.2270x over previous
import functools

import jax
import jax.numpy as jnp
from jax import lax
from jax.experimental import pallas as pl
from jax.experimental.pallas import tpu as pltpu
from jax.experimental.pallas import tpu_sc as plsc

INPUT_DIM = 1000000
EMBED_DIM = 32
B = 16384
T = 50
TPAD = 64

NC, NS = 2, 16
NW = NC * NS
B_PER_W = B // NW
NB = 16
NBLK = B_PER_W // NB
NPAIR = NBLK // 2


def _make_kernel():
    mesh = plsc.VectorSubcoreMesh(core_axis_name="c", subcore_axis_name="s")

    @functools.partial(
        pl.kernel,
        out_type=jax.ShapeDtypeStruct((B, TPAD, EMBED_DIM), jnp.float32),
        mesh=mesh,
        scratch_types=[
            pltpu.VMEM((B_PER_W, T), jnp.int32),
            pltpu.VMEM((NB * T, EMBED_DIM), jnp.float32),
            pltpu.VMEM((NB * T, EMBED_DIM), jnp.float32),
            pltpu.SemaphoreType.DMA,
            pltpu.SemaphoreType.DMA,
        ],
        compiler_params=pltpu.CompilerParams(use_tc_tiling_on_sc=False),
    )
    def emb(x_hbm, table_hbm, out_hbm, idx_v, bufA, bufB, gsem, ssem):
        wid = lax.axis_index("s") * NC + lax.axis_index("c")
        base_b = wid * B_PER_W
        pltpu.sync_copy(x_hbm.at[pl.ds(base_b, B_PER_W)], idx_v)

        def fire_gathers(blk, buf):
            @pl.loop(0, NB)
            def _(k):
                pltpu.make_async_copy(
                    table_hbm.at[idx_v.at[blk * NB + k]],
                    buf.at[pl.ds(k * T, T)],
                    gsem,
                ).start()

        def drain_gathers_fire_writes(blk, buf):
            @pl.loop(0, NB)
            def _(k):
                pltpu.make_async_copy(
                    table_hbm.at[pl.ds(0, T)],
                    buf.at[pl.ds(k * T, T)],
                    gsem,
                ).wait()
                pltpu.make_async_copy(
                    buf.at[pl.ds(k * T, T)],
                    out_hbm.at[base_b + blk * NB + k, pl.ds(0, T)],
                    ssem,
                ).start()

        def wait_writes(blk, buf):
            @pl.loop(0, NB)
            def _(k):
                pltpu.make_async_copy(
                    buf.at[pl.ds(k * T, T)],
                    out_hbm.at[base_b + blk * NB + k, pl.ds(0, T)],
                    ssem,
                ).wait()

        fire_gathers(0, bufA)

        @pl.loop(0, NPAIR)
        def _(p):
            blk0 = 2 * p

            @pl.when(p > 0)
            def _():
                wait_writes(blk0 - 1, bufB)

            fire_gathers(blk0 + 1, bufB)
            drain_gathers_fire_writes(blk0, bufA)
            wait_writes(blk0, bufA)

            @pl.when(p < NPAIR - 1)
            def _():
                fire_gathers(blk0 + 2, bufA)

            drain_gathers_fire_writes(blk0 + 1, bufB)

        wait_writes(NBLK - 1, bufB)

    return emb


_emb_kernel = _make_kernel()


def kernel(x, wts):
    out = _emb_kernel(x, wts)
    return lax.slice(out, (0, 0, 0), (B, T, EMBED_DIM))


# speedup vs baseline: 1.6248x; 1.1896x over previous
import functools

import jax
import jax.numpy as jnp
from jax import lax
from jax.experimental import pallas as pl
from jax.experimental.pallas import tpu as pltpu
from jax.experimental.pallas import tpu_sc as plsc

INPUT_DIM = 1000000
EMBED_DIM = 32
B = 16384
T = 50

NC, NS = 2, 16
NW = NC * NS
BT = 128
ET = 8
K = 4
B_PER_W = K * BT
NBT = B // BT


def _make_kernel():
    mesh = plsc.VectorSubcoreMesh(core_axis_name="c", subcore_axis_name="s")

    @functools.partial(
        pl.kernel,
        out_type=jax.ShapeDtypeStruct((T, EMBED_DIM // ET, NBT, ET, BT),
                                      jnp.float32),
        mesh=mesh,
        scratch_types=[
            pltpu.VMEM((B_PER_W, T), jnp.int32),
            pltpu.VMEM((T, K, BT), jnp.int32),
            pltpu.VMEM((K, BT, EMBED_DIM), jnp.float32),
            pltpu.VMEM((2, EMBED_DIM // ET, ET, BT), jnp.float32),
            pltpu.SemaphoreType.DMA,
            pltpu.SemaphoreType.DMA,
        ],
        compiler_params=pltpu.CompilerParams(use_tc_tiling_on_sc=False,
                                             needs_layout_passes=False),
    )
    def emb(x_hbm, table_hbm, out_hbm, idx_v, idxT_v, gbufs, sbufs,
            gsem, wsem):
        wid = lax.axis_index("s") * NC + lax.axis_index("c")
        base_b = wid * B_PER_W
        iota = jax.lax.iota(jnp.int32, 16)

        pltpu.sync_copy(x_hbm.at[pl.ds(base_b, B_PER_W)], idx_v)

        @pl.loop(0, T)
        def _(t):
            tcol = jnp.full((16,), t, jnp.int32)
            for k in range(K):
                for g in range(BT // 16):
                    rows = iota + (k * BT + g * 16)
                    v = plsc.load_gather(idx_v, [rows, tcol])
                    idxT_v[t, k, pl.ds(g * 16, 16)] = v

        def gstart(t, k):
            pltpu.make_async_copy(
                table_hbm.at[idxT_v.at[t, k]], gbufs.at[k], gsem).start()

        def gwait(k):
            pltpu.make_async_copy(
                table_hbm.at[pl.ds(0, BT)], gbufs.at[k], gsem).wait()

        def wdesc(t, k, s, et):
            return pltpu.make_async_copy(
                sbufs.at[s, et], out_hbm.at[t, et, wid * K + k], wsem)

        for k in range(K):
            gstart(0, k)

        @pl.loop(0, T)
        def _(t):
            for k in range(K):
                s = k % 2
                gwait(k)
                if k < 2:
                    @pl.when(t > 0)
                    def _():
                        for et in range(EMBED_DIM // ET):
                            wdesc(t - 1, k + 2, s, et).wait()
                else:
                    for et in range(EMBED_DIM // ET):
                        wdesc(t, k - 2, s, et).wait()

                @pl.loop(0, BT // 16)
                def _(g):
                    rows = iota + g * 16
                    for e in range(EMBED_DIM):
                        ecol = jnp.full((16,), e, jnp.int32)
                        v = plsc.load_gather(gbufs.at[k], [rows, ecol])
                        sbufs[s, e // ET, e % ET, pl.ds(g * 16, 16)] = v

                for et in range(EMBED_DIM // ET):
                    wdesc(t, k, s, et).start()

                @pl.when(t < T - 1)
                def _():
                    gstart(t + 1, k)

        for k in (2, 3):
            for et in range(EMBED_DIM // ET):
                wdesc(T - 1, k, k % 2, et).wait()

    return emb


_emb_kernel = _make_kernel()


def kernel(x, wts):
    out5 = _emb_kernel(x, wts)
    return out5.transpose(2, 4, 0, 1, 3).reshape(B, T, EMBED_DIM)


# speedup vs baseline: 2.6102x; 1.6065x over previous
import functools

import jax
import jax.numpy as jnp
from jax import lax
from jax.experimental import pallas as pl
from jax.experimental.pallas import tpu as pltpu
from jax.experimental.pallas import tpu_sc as plsc

INPUT_DIM = 1000000
EMBED_DIM = 32
B = 16384
T = 50

NC, NS = 2, 16
NW = NC * NS
BT = 128
ET = 8
K = 4
B_PER_W = K * BT
NBT = B // BT


def _make_kernel():
    mesh = plsc.VectorSubcoreMesh(core_axis_name="c", subcore_axis_name="s")

    @functools.partial(
        pl.kernel,
        out_type=jax.ShapeDtypeStruct((T, EMBED_DIM // ET, NBT, ET, BT),
                                      jnp.float32),
        mesh=mesh,
        scratch_types=[
            pltpu.VMEM((B_PER_W, T), jnp.int32),
            pltpu.VMEM((T, K, BT), jnp.int32),
            pltpu.VMEM((K, BT, EMBED_DIM), jnp.float32),
            pltpu.VMEM((2, EMBED_DIM, BT + 1), jnp.float32),
            pltpu.SemaphoreType.DMA,
            pltpu.SemaphoreType.DMA,
        ],
        compiler_params=pltpu.CompilerParams(use_tc_tiling_on_sc=False,
                                             needs_layout_passes=False),
    )
    def emb(x_hbm, table_hbm, out_hbm, idx_v, idxT_v, gbufs, sbufs,
            gsem, wsem):
        wid = lax.axis_index("s") * NC + lax.axis_index("c")
        base_b = wid * B_PER_W
        iota = jax.lax.iota(jnp.int32, 16)
        erows = [iota + h * 16 for h in range(2)]

        pltpu.sync_copy(x_hbm.at[pl.ds(base_b, B_PER_W)], idx_v)

        @pl.loop(0, T)
        def _(t):
            tcol = jnp.full((16,), t, jnp.int32)
            for k in range(K):
                for g in range(BT // 16):
                    rows = iota + (k * BT + g * 16)
                    v = plsc.load_gather(idx_v, [rows, tcol])
                    idxT_v[t, k, pl.ds(g * 16, 16)] = v

        def gstart(t, k):
            pltpu.make_async_copy(
                table_hbm.at[idxT_v.at[t, k]], gbufs.at[k], gsem).start()

        def gwait(k):
            pltpu.make_async_copy(
                table_hbm.at[pl.ds(0, BT)], gbufs.at[k], gsem).wait()

        def wdesc(t, k, s, et):
            return pltpu.make_async_copy(
                sbufs.at[s, pl.ds(et * ET, ET), pl.ds(0, BT)],
                out_hbm.at[t, et, wid * K + k], wsem)

        for k in range(K):
            gstart(0, k)

        @pl.loop(0, T)
        def _(t):
            for k in range(K):
                s = k % 2
                gwait(k)
                if k < 2:
                    @pl.when(t > 0)
                    def _():
                        for et in range(EMBED_DIM // ET):
                            wdesc(t - 1, k + 2, s, et).wait()
                else:
                    for et in range(EMBED_DIM // ET):
                        wdesc(t, k - 2, s, et).wait()

                @pl.loop(0, BT, unroll=8)
                def _(b):
                    bb = jnp.full((16,), b, jnp.int32)
                    for h in range(2):
                        v = gbufs[k, b, pl.ds(h * 16, 16)]
                        plsc.store_scatter(sbufs.at[s], [erows[h], bb], v)

                for et in range(EMBED_DIM // ET):
                    wdesc(t, k, s, et).start()

                @pl.when(t < T - 1)
                def _():
                    gstart(t + 1, k)

        for k in (2, 3):
            for et in range(EMBED_DIM // ET):
                wdesc(T - 1, k, k % 2, et).wait()

    return emb


_emb_kernel = _make_kernel()


def kernel(x, wts):
    out5 = _emb_kernel(x, wts)
    return out5.transpose(2, 4, 0, 1, 3).reshape(B, T, EMBED_DIM)
